# baseline (speedup 1.0000x reference)
import functools

import jax
import jax.numpy as jnp
from jax import lax
from jax.experimental import pallas as pl
from jax.experimental.pallas import tpu as pltpu
from jax.experimental.pallas import tpu_sc as plsc

ROWS = 400
COLS = 500
D = 128
L = 16
H1 = 248
H2 = COLS - H1
NC, NS = 2, 16
NW = NC * NS
KFULL = ROWS // NW
EXTRA = ROWS - KFULL * NW
TABLE_ROWS = 50


def _body(x3, table1, pos2, out3,
          tbuf, xa, xb, p0, p1,
          stb, sxa, sxb, sp0, sp1, soa, sob):
    wid = lax.axis_index("s") * NC + lax.axis_index("c")

    pltpu.async_copy(table1, tbuf, stb).wait()
    cols = [lax.iota(jnp.int32, L) + j * L for j in range(D // L)]

    def rowid(k):
        return k * NW + wid

    def issue_xa(k):
        pltpu.async_copy(x3.at[rowid(k), pl.ds(0, H1)], xa, sxa)

    def issue_xb(k):
        pltpu.async_copy(x3.at[rowid(k), pl.ds(H1, H2)], xb, sxb)

    def wait_xa():
        pltpu.make_async_copy(x3.at[0, pl.ds(0, H1)], xa, sxa).wait()

    def wait_xb():
        pltpu.make_async_copy(x3.at[0, pl.ds(H1, H2)], xb, sxb).wait()

    def issue_p(k, pref, sp):
        pltpu.async_copy(pos2.at[rowid(k)], pref, sp)

    def wait_p(pref, sp):
        pltpu.make_async_copy(pos2.at[0], pref, sp).wait()

    def issue_oa(k):
        pltpu.async_copy(xa, out3.at[rowid(k), pl.ds(0, H1)], soa)

    def issue_ob(k):
        pltpu.async_copy(xb, out3.at[rowid(k), pl.ds(H1, H2)], sob)

    def wait_oa():
        pltpu.make_async_copy(xa, out3.at[0, pl.ds(0, H1)], soa).wait()

    def wait_ob():
        pltpu.make_async_copy(xb, out3.at[0, pl.ds(H1, H2)], sob).wait()

    def valid(k):
        return (k < KFULL) | ((k == KFULL) & (wid < EXTRA))

    def add_half(xref, pref, ntok, off):
        @plsc.parallel_loop(0, ntok, unroll=16)
        def _tok(t):
            posv = plsc.load_gather(pref, [jnp.full((L,), t + off, jnp.int32)])
            base = posv * D
            for j in range(D // L):
                rows = plsc.load_gather(tbuf, [base + cols[j]])
                plsc.addupdate(xref.at[t, pl.ds(j * L, L)], rows)

    issue_p(0, p0, sp0)
    issue_p(1, p1, sp1)
    issue_xa(0)
    issue_xb(0)

    def pair_body(p, carry):
        k0 = 2 * p
        k1 = k0 + 1
        wait_p(p0, sp0)
        wait_xa()
        add_half(xa, p0, H1, 0)
        issue_oa(k0)
        wait_xb()
        add_half(xb, p0, H2, H1)
        issue_ob(k0)

        @pl.when(valid(k0 + 2))
        def _():
            issue_p(k0 + 2, p0, sp0)

        wait_oa()
        issue_xa(k1)
        wait_p(p1, sp1)
        wait_xa()
        add_half(xa, p1, H1, 0)
        issue_oa(k1)
        wait_ob()
        issue_xb(k1)
        wait_xb()
        add_half(xb, p1, H2, H1)
        issue_ob(k1)

        @pl.when(valid(k1 + 2))
        def _():
            issue_p(k1 + 2, p1, sp1)

        @pl.when(valid(k0 + 2))
        def _():
            wait_oa()
            issue_xa(k0 + 2)
            wait_ob()
            issue_xb(k0 + 2)
        return carry

    lax.fori_loop(0, KFULL // 2, pair_body, 0)

    @pl.when(wid < EXTRA)
    def _extra():
        wait_p(p0, sp0)
        wait_xa()
        add_half(xa, p0, H1, 0)
        issue_oa(KFULL)
        wait_xb()
        add_half(xb, p0, H2, H1)
        issue_ob(KFULL)

    wait_oa()
    wait_ob()


@jax.jit
def kernel(x, emb_table, pos_matrix):
    table1 = emb_table.reshape(TABLE_ROWS * D)
    mesh = plsc.VectorSubcoreMesh(core_axis_name="c", subcore_axis_name="s")
    run = functools.partial(
        pl.kernel,
        mesh=mesh,
        out_type=jax.ShapeDtypeStruct((ROWS, COLS, D), jnp.float32),
        compiler_params=pltpu.CompilerParams(
            needs_layout_passes=False, use_tc_tiling_on_sc=True),
        scratch_types=[
            pltpu.VMEM((TABLE_ROWS * D,), jnp.float32),
            pltpu.VMEM((H1, D), jnp.float32),
            pltpu.VMEM((H2, D), jnp.float32),
            pltpu.VMEM((COLS,), jnp.int32),
            pltpu.VMEM((COLS,), jnp.int32),
        ] + [pltpu.SemaphoreType.DMA] * 7,
    )(_body)
    return run(x, table1, pos_matrix)


# speedup vs baseline: 1.0429x; 1.0429x over previous
import functools

import jax
import jax.numpy as jnp
from jax import lax
from jax.experimental import pallas as pl
from jax.experimental.pallas import tpu as pltpu
from jax.experimental.pallas import tpu_sc as plsc

ROWS = 400
COLS = 500
D = 128
L = 16
H1 = 248
H2 = COLS - H1
NC, NS = 2, 16
NW = NC * NS
KFULL = ROWS // NW
EXTRA = ROWS - KFULL * NW
TABLE_ROWS = 50


def _body(x3, table1, pos2, out3,
          tbuf, xa, xb, p0, p1,
          stb, sxa, sxb, sp0, sp1, soa, sob):
    wid = lax.axis_index("s") * NC + lax.axis_index("c")

    pltpu.async_copy(table1, tbuf, stb).wait()
    cols = [lax.iota(jnp.int32, L) + j * L for j in range(D // L)]

    def rowid(k):
        return k * NW + wid

    def issue_xa(k):
        pltpu.async_copy(x3.at[rowid(k), pl.ds(0, H1)], xa, sxa)

    def issue_xb(k):
        pltpu.async_copy(x3.at[rowid(k), pl.ds(H1, H2)], xb, sxb)

    def wait_xa():
        pltpu.make_async_copy(x3.at[0, pl.ds(0, H1)], xa, sxa).wait()

    def wait_xb():
        pltpu.make_async_copy(x3.at[0, pl.ds(H1, H2)], xb, sxb).wait()

    def issue_p(k, pref, sp):
        pltpu.async_copy(pos2.at[rowid(k)], pref, sp)

    def wait_p(pref, sp):
        pltpu.make_async_copy(pos2.at[0], pref, sp).wait()

    def issue_oa(k):
        pltpu.async_copy(xa, out3.at[rowid(k), pl.ds(0, H1)], soa)

    def issue_ob(k):
        pltpu.async_copy(xb, out3.at[rowid(k), pl.ds(H1, H2)], sob)

    def wait_oa():
        pltpu.make_async_copy(xa, out3.at[0, pl.ds(0, H1)], soa).wait()

    def wait_ob():
        pltpu.make_async_copy(xb, out3.at[0, pl.ds(H1, H2)], sob).wait()

    def valid(k):
        return (k < KFULL) | ((k == KFULL) & (wid < EXTRA))

    def add_half(xref, pref, ntok, off):
        @plsc.parallel_loop(0, ntok, unroll=4)
        def _tok(t):
            posv = plsc.load_gather(pref, [jnp.full((L,), t + off, jnp.int32)])
            base = posv * D
            for j in range(D // L):
                rows = plsc.load_gather(tbuf, [base + cols[j]])
                plsc.addupdate(xref.at[t, pl.ds(j * L, L)], rows)

    issue_p(0, p0, sp0)
    issue_p(1, p1, sp1)
    issue_xa(0)
    issue_xb(0)

    def pair_body(p, carry):
        k0 = 2 * p
        k1 = k0 + 1
        wait_p(p0, sp0)
        wait_xa()
        add_half(xa, p0, H1, 0)
        issue_oa(k0)
        wait_xb()
        add_half(xb, p0, H2, H1)
        issue_ob(k0)

        @pl.when(valid(k0 + 2))
        def _():
            issue_p(k0 + 2, p0, sp0)

        wait_oa()
        issue_xa(k1)
        wait_p(p1, sp1)
        wait_xa()
        add_half(xa, p1, H1, 0)
        issue_oa(k1)
        wait_ob()
        issue_xb(k1)
        wait_xb()
        add_half(xb, p1, H2, H1)
        issue_ob(k1)

        @pl.when(valid(k1 + 2))
        def _():
            issue_p(k1 + 2, p1, sp1)

        @pl.when(valid(k0 + 2))
        def _():
            wait_oa()
            issue_xa(k0 + 2)
            wait_ob()
            issue_xb(k0 + 2)
        return carry

    lax.fori_loop(0, KFULL // 2, pair_body, 0)

    @pl.when(wid < EXTRA)
    def _extra():
        wait_p(p0, sp0)
        wait_xa()
        add_half(xa, p0, H1, 0)
        issue_oa(KFULL)
        wait_xb()
        add_half(xb, p0, H2, H1)
        issue_ob(KFULL)

    wait_oa()
    wait_ob()


@jax.jit
def kernel(x, emb_table, pos_matrix):
    table1 = emb_table.reshape(TABLE_ROWS * D)
    mesh = plsc.VectorSubcoreMesh(core_axis_name="c", subcore_axis_name="s")
    run = functools.partial(
        pl.kernel,
        mesh=mesh,
        out_type=jax.ShapeDtypeStruct((ROWS, COLS, D), jnp.float32),
        compiler_params=pltpu.CompilerParams(
            needs_layout_passes=False, use_tc_tiling_on_sc=True),
        scratch_types=[
            pltpu.VMEM((TABLE_ROWS * D,), jnp.float32),
            pltpu.VMEM((H1, D), jnp.float32),
            pltpu.VMEM((H2, D), jnp.float32),
            pltpu.VMEM((COLS,), jnp.int32),
            pltpu.VMEM((COLS,), jnp.int32),
        ] + [pltpu.SemaphoreType.DMA] * 7,
    )(_body)
    return run(x, table1, pos_matrix)


# speedup vs baseline: 1.0450x; 1.0020x over previous
import functools

import jax
import jax.numpy as jnp
from jax import lax
from jax.experimental import pallas as pl
from jax.experimental.pallas import tpu as pltpu
from jax.experimental.pallas import tpu_sc as plsc

ROWS = 400
COLS = 500
D = 128
L = 16
H1 = 248
H2 = COLS - H1
NC, NS = 2, 16
NW = NC * NS
KFULL = ROWS // NW
EXTRA = ROWS - KFULL * NW
TABLE_ROWS = 50


def _body(x3, table1, pos2, out3,
          tbuf, xa, xb, p0, p1,
          stb, sxa, sxb, sp0, sp1, soa, sob):
    wid = lax.axis_index("s") * NC + lax.axis_index("c")

    pltpu.async_copy(table1, tbuf, stb).wait()
    cols = [lax.iota(jnp.int32, L) + j * L for j in range(D // L)]

    def rowid(k):
        return k * NW + wid

    def issue_xa(k):
        pltpu.async_copy(x3.at[rowid(k), pl.ds(0, H1)], xa, sxa)

    def issue_xb(k):
        pltpu.async_copy(x3.at[rowid(k), pl.ds(H1, H2)], xb, sxb)

    def wait_xa():
        pltpu.make_async_copy(x3.at[0, pl.ds(0, H1)], xa, sxa).wait()

    def wait_xb():
        pltpu.make_async_copy(x3.at[0, pl.ds(H1, H2)], xb, sxb).wait()

    def issue_p(k, pref, sp):
        pltpu.async_copy(pos2.at[rowid(k)], pref, sp)

    def wait_p(pref, sp):
        pltpu.make_async_copy(pos2.at[0], pref, sp).wait()

    def issue_oa(k):
        pltpu.async_copy(xa, out3.at[rowid(k), pl.ds(0, H1)], soa)

    def issue_ob(k):
        pltpu.async_copy(xb, out3.at[rowid(k), pl.ds(H1, H2)], sob)

    def wait_oa():
        pltpu.make_async_copy(xa, out3.at[0, pl.ds(0, H1)], soa).wait()

    def wait_ob():
        pltpu.make_async_copy(xb, out3.at[0, pl.ds(H1, H2)], sob).wait()

    def valid(k):
        return (k < KFULL) | ((k == KFULL) & (wid < EXTRA))

    def add_half(xref, pref, ntok, off):
        @plsc.parallel_loop(0, ntok, unroll=2)
        def _tok(t):
            posv = plsc.load_gather(pref, [jnp.full((L,), t + off, jnp.int32)])
            base = posv * D
            for j in range(D // L):
                rows = plsc.load_gather(tbuf, [base + cols[j]])
                plsc.addupdate(xref.at[t, pl.ds(j * L, L)], rows)

    issue_p(0, p0, sp0)
    issue_p(1, p1, sp1)
    issue_xa(0)
    issue_xb(0)

    def pair_body(p, carry):
        k0 = 2 * p
        k1 = k0 + 1
        wait_p(p0, sp0)
        wait_xa()
        add_half(xa, p0, H1, 0)
        issue_oa(k0)
        wait_xb()
        add_half(xb, p0, H2, H1)
        issue_ob(k0)

        @pl.when(valid(k0 + 2))
        def _():
            issue_p(k0 + 2, p0, sp0)

        wait_oa()
        issue_xa(k1)
        wait_p(p1, sp1)
        wait_xa()
        add_half(xa, p1, H1, 0)
        issue_oa(k1)
        wait_ob()
        issue_xb(k1)
        wait_xb()
        add_half(xb, p1, H2, H1)
        issue_ob(k1)

        @pl.when(valid(k1 + 2))
        def _():
            issue_p(k1 + 2, p1, sp1)

        @pl.when(valid(k0 + 2))
        def _():
            wait_oa()
            issue_xa(k0 + 2)
            wait_ob()
            issue_xb(k0 + 2)
        return carry

    lax.fori_loop(0, KFULL // 2, pair_body, 0)

    @pl.when(wid < EXTRA)
    def _extra():
        wait_p(p0, sp0)
        wait_xa()
        add_half(xa, p0, H1, 0)
        issue_oa(KFULL)
        wait_xb()
        add_half(xb, p0, H2, H1)
        issue_ob(KFULL)

    wait_oa()
    wait_ob()


@jax.jit
def kernel(x, emb_table, pos_matrix):
    table1 = emb_table.reshape(TABLE_ROWS * D)
    mesh = plsc.VectorSubcoreMesh(core_axis_name="c", subcore_axis_name="s")
    run = functools.partial(
        pl.kernel,
        mesh=mesh,
        out_type=jax.ShapeDtypeStruct((ROWS, COLS, D), jnp.float32),
        compiler_params=pltpu.CompilerParams(
            needs_layout_passes=False, use_tc_tiling_on_sc=True),
        scratch_types=[
            pltpu.VMEM((TABLE_ROWS * D,), jnp.float32),
            pltpu.VMEM((H1, D), jnp.float32),
            pltpu.VMEM((H2, D), jnp.float32),
            pltpu.VMEM((COLS,), jnp.int32),
            pltpu.VMEM((COLS,), jnp.int32),
        ] + [pltpu.SemaphoreType.DMA] * 7,
    )(_body)
    return run(x, table1, pos_matrix)
